# baseline (speedup 1.0000x reference)
import functools

import jax
import jax.numpy as jnp
from jax import lax
from jax.experimental import pallas as pl
from jax.experimental.pallas import tpu as pltpu
from jax.experimental.pallas import tpu_sc as plsc

D = 64
KP1 = 21
NC = 2
NS = 16
NW = NC * NS
CH = 64
IDX_CHUNK = 112


def _dot_kernel(cw_hbm, cat_hbm, ctab_hbm, xtab_hbm, pos_hbm, neg_hbm,
                cidx_v, catidx_v, crow_v, catrow_v, pos_v, nout_v, sem,
                *, b_per_w, n_chunks, n_neg):
    wid = lax.axis_index("s") * NC + lax.axis_index("c")
    base = wid * b_per_w
    n_gathers = (CH * KP1) // IDX_CHUNK

    def chunk_body(c, carry):
        cs = base + c * CH
        pltpu.sync_copy(cw_hbm.at[pl.ds(cs, CH)], cidx_v)
        pltpu.sync_copy(cat_hbm.at[pl.ds(cs * KP1, CH * KP1)], catidx_v)
        copies = [pltpu.async_copy(ctab_hbm.at[cidx_v], crow_v, sem)]
        for j in range(n_gathers):
            copies.append(pltpu.async_copy(
                xtab_hbm.at[catidx_v.at[pl.ds(j * IDX_CHUNK, IDX_CHUNK)]],
                catrow_v.at[pl.ds(j * IDX_CHUNK, IDX_CHUNK)],
                sem))
        for cp in copies:
            cp.wait()

        def group_body(g, carry2):
            lane = lax.broadcasted_iota(jnp.int32, (16,), 0)
            bvec = g * 16 + lane
            cat_rows = bvec * KP1
            acc_pos = jnp.zeros((16,), jnp.float32)
            accs = [jnp.zeros((16,), jnp.float32) for _ in range(n_neg)]
            for db in range(4):
                cregs = []
                for dd in range(16):
                    dsp = jnp.full((16,), db * 16 + dd, jnp.int32)
                    cregs.append(plsc.load_gather(crow_v, [bvec, dsp]))
                for dd in range(16):
                    dsp = jnp.full((16,), db * 16 + dd, jnp.int32)
                    xv = plsc.load_gather(catrow_v, [cat_rows, dsp])
                    acc_pos = acc_pos + cregs[dd] * xv
                for k in range(n_neg):
                    rowk = cat_rows + (k + 1)
                    for dd in range(16):
                        dsp = jnp.full((16,), db * 16 + dd, jnp.int32)
                        nv = plsc.load_gather(catrow_v, [rowk, dsp])
                        accs[k] = accs[k] + cregs[dd] * nv
            pos_v[pl.ds(g * 16, 16)] = acc_pos
            sbase = bvec * n_neg
            for k in range(n_neg):
                plsc.store_scatter(nout_v, [sbase + k], -accs[k])
            return carry2

        lax.fori_loop(0, CH // 16, group_body, 0)
        pltpu.sync_copy(pos_v, pos_hbm.at[pl.ds(cs, CH)])
        pltpu.sync_copy(nout_v, neg_hbm.at[pl.ds(cs * n_neg, CH * n_neg)])
        return carry

    lax.fori_loop(0, n_chunks, chunk_body, 0)


@jax.jit
def kernel(center_words, context_words, neg_samples, center_table, context_table):
    B, K = neg_samples.shape
    b_per_w = B // NW
    n_chunks = b_per_w // CH
    cw = center_words.astype(jnp.int32)
    cat = jnp.concatenate(
        [context_words.astype(jnp.int32)[:, None], neg_samples.astype(jnp.int32)],
        axis=1).reshape(B * KP1)

    mesh = plsc.VectorSubcoreMesh(core_axis_name="c", subcore_axis_name="s")
    run = pl.kernel(
        functools.partial(_dot_kernel, b_per_w=b_per_w, n_chunks=n_chunks,
                          n_neg=K),
        out_type=[
            jax.ShapeDtypeStruct((B,), jnp.float32),
            jax.ShapeDtypeStruct((B * K,), jnp.float32),
        ],
        mesh=mesh,
        compiler_params=pltpu.CompilerParams(needs_layout_passes=False,
                                             use_tc_tiling_on_sc=False),
        scratch_types=[
            pltpu.VMEM((CH,), jnp.int32),
            pltpu.VMEM((CH * KP1,), jnp.int32),
            pltpu.VMEM((CH, D), jnp.float32),
            pltpu.VMEM((CH * KP1, D), jnp.float32),
            pltpu.VMEM((CH,), jnp.float32),
            pltpu.VMEM((CH * K,), jnp.float32),
            pltpu.SemaphoreType.DMA,
        ],
    )
    pos, neg = run(cw, cat, center_table, context_table)
    return pos, neg.reshape(B, K)
